# baseline (speedup 1.0000x reference)
import jax
import jax.numpy as jnp
from jax.experimental import pallas as pl
from jax.experimental.pallas import tpu as pltpu

N = 4096
IN_FEAT = 128
HID = 64
LAT = 16
SCALE = 1.8
BM = 128
NBLK = N // BM
EPS = 1e-12


def _body(a_ref, x_ref, w1_ref, w2_ref, o_ref,
          a16_s, d_s, zd1_s, zd2_s, enc_s):
    i = pl.program_id(0)
    phase = i // NBLK
    r = i % NBLK
    rows = pl.ds(r * BM, BM)

    @pl.when(phase == 0)
    def _rowsum_and_cast():
        a_blk = a_ref[...]
        s = jnp.sum(a_blk, axis=1, keepdims=True) + 1.0
        d_s[rows, :] = jax.lax.rsqrt(s)
        a16_s[rows, :] = a_blk.astype(jnp.bfloat16)

    @pl.when(i == NBLK)
    def _prep():
        z = jnp.dot(x_ref[...], w1_ref[...], preferred_element_type=jnp.float32)
        n = jnp.sqrt(jnp.sum(z * z, axis=1, keepdims=True))
        zd1_s[...] = d_s[...] * (z / jnp.maximum(n, EPS))

    @pl.when(phase == 1)
    def _conv1():
        d_blk = d_s[rows, :]
        acc = jnp.dot(a16_s[rows, :], zd1_s[...].astype(jnp.bfloat16),
                      preferred_element_type=jnp.float32)
        h = jnp.maximum(SCALE * d_blk * (acc + zd1_s[rows, :]), 0.0)
        g = jnp.dot(h, w2_ref[...], preferred_element_type=jnp.float32)
        n = jnp.sqrt(jnp.sum(g * g, axis=1, keepdims=True))
        zd2_s[rows, :] = d_blk * (g / jnp.maximum(n, EPS))

    @pl.when(phase == 2)
    def _conv2():
        d_blk = d_s[rows, :]
        acc = jnp.dot(a16_s[rows, :], zd2_s[...].astype(jnp.bfloat16),
                      preferred_element_type=jnp.float32)
        enc_s[rows, :] = SCALE * d_blk * (acc + zd2_s[rows, :])

    @pl.when(phase == 3)
    def _outer():
        p = jax.lax.dot_general(
            enc_s[rows, :], enc_s[...],
            (((1,), (1,)), ((), ())),
            preferred_element_type=jnp.float32,
        )
        o_ref[...] = 0.5 * jnp.tanh(0.5 * p) + 0.5


def kernel(A, X, W1, W2):
    return pl.pallas_call(
        _body,
        grid=(4 * NBLK,),
        in_specs=[
            pl.BlockSpec((BM, N), lambda i: (jnp.where(i < NBLK, i, NBLK - 1), 0)),
            pl.BlockSpec((N, IN_FEAT), lambda i: (0, 0)),
            pl.BlockSpec((IN_FEAT, HID), lambda i: (0, 0)),
            pl.BlockSpec((HID, LAT), lambda i: (0, 0)),
        ],
        out_specs=pl.BlockSpec(
            (BM, N), lambda i: (jnp.where(i >= 3 * NBLK, i % NBLK, 0), 0)
        ),
        out_shape=jax.ShapeDtypeStruct((N, N), jnp.float32),
        scratch_shapes=[
            pltpu.VMEM((N, N), jnp.bfloat16),
            pltpu.VMEM((N, 1), jnp.float32),
            pltpu.VMEM((N, HID), jnp.float32),
            pltpu.VMEM((N, LAT), jnp.float32),
            pltpu.VMEM((N, LAT), jnp.float32),
        ],
        compiler_params=pltpu.CompilerParams(
            dimension_semantics=("arbitrary",),
        ),
    )(A, X, W1, W2)


# speedup vs baseline: 1.3081x; 1.3081x over previous
import jax
import jax.numpy as jnp
from jax.experimental import pallas as pl
from jax.experimental.pallas import tpu as pltpu

N = 4096
IN_FEAT = 128
HID = 64
LAT = 16
SCALE = 1.8
BM = 256
NBLK = N // BM
EPS = 1e-12


def _body(a_ref, x_ref, w1_ref, w2_ref, o_ref, a16_s, ws, wb):
    i = pl.program_id(0)
    phase = i // NBLK
    r = i % NBLK
    rows = pl.ds(r * BM, BM)

    @pl.when(phase == 0)
    def _rowsum_cast_prep():
        a_blk = a_ref[...]
        s = jnp.sum(a_blk, axis=1, keepdims=True) + 1.0
        d_blk = jax.lax.rsqrt(s)
        ws[rows, 96:97] = d_blk
        a16_s[rows, :] = a_blk.astype(jnp.bfloat16)
        z = jnp.dot(x_ref[...], w1_ref[...], preferred_element_type=jnp.float32)
        n = jnp.sqrt(jnp.sum(z * z, axis=1, keepdims=True))
        zd1 = d_blk * (z / jnp.maximum(n, EPS))
        ws[rows, 0:64] = zd1
        wb[rows, 0:64] = zd1.astype(jnp.bfloat16)

    @pl.when(phase == 1)
    def _conv1():
        d_blk = ws[rows, 96:97]
        acc = jnp.dot(a16_s[rows, :], wb[:, 0:64],
                      preferred_element_type=jnp.float32)
        h = jnp.maximum(SCALE * d_blk * (acc + ws[rows, 0:64]), 0.0)
        g = jnp.dot(h, w2_ref[...], preferred_element_type=jnp.float32)
        n = jnp.sqrt(jnp.sum(g * g, axis=1, keepdims=True))
        zd2 = d_blk * (g / jnp.maximum(n, EPS))
        ws[rows, 64:80] = zd2
        wb[rows, 64:80] = zd2.astype(jnp.bfloat16)

    @pl.when(phase == 2)
    def _conv2():
        d_blk = ws[rows, 96:97]
        acc = jnp.dot(a16_s[rows, :], wb[:, 64:80],
                      preferred_element_type=jnp.float32)
        ws[rows, 80:96] = SCALE * d_blk * (acc + ws[rows, 64:80])

    @pl.when(phase == 3)
    def _outer():
        p = jax.lax.dot_general(
            ws[rows, 80:96], ws[:, 80:96],
            (((1,), (1,)), ((), ())),
            preferred_element_type=jnp.float32,
        )
        o_ref[...] = 0.5 * jnp.tanh(0.5 * p) + 0.5


def kernel(A, X, W1, W2):
    return pl.pallas_call(
        _body,
        grid=(4 * NBLK,),
        in_specs=[
            pl.BlockSpec((BM, N), lambda i: (jnp.where(i < NBLK, i, NBLK - 1), 0)),
            pl.BlockSpec((BM, IN_FEAT), lambda i: (jnp.where(i < NBLK, i, NBLK - 1), 0)),
            pl.BlockSpec((IN_FEAT, HID), lambda i: (0, 0)),
            pl.BlockSpec((HID, LAT), lambda i: (0, 0)),
        ],
        out_specs=pl.BlockSpec(
            (BM, N), lambda i: (jnp.where(i >= 3 * NBLK, i % NBLK, 0), 0)
        ),
        out_shape=jax.ShapeDtypeStruct((N, N), jnp.float32),
        scratch_shapes=[
            pltpu.VMEM((N, N), jnp.bfloat16),
            pltpu.VMEM((N, 128), jnp.float32),
            pltpu.VMEM((N, 128), jnp.bfloat16),
        ],
        compiler_params=pltpu.CompilerParams(
            dimension_semantics=("arbitrary",),
        ),
    )(A, X, W1, W2)


# speedup vs baseline: 1.3563x; 1.0369x over previous
import jax
import jax.numpy as jnp
from jax.experimental import pallas as pl
from jax.experimental.pallas import tpu as pltpu

N = 4096
IN_FEAT = 128
HID = 64
LAT = 16
SCALE = 1.8
BM = 256
NBLK = N // BM
EPS = 1e-12


def _body(a_ref, x_ref, w1_ref, w2_ref, o_ref, a16_s, ws, wb, acc_s):
    i = pl.program_id(0)
    phase = i // NBLK
    r = i % NBLK
    rows = pl.ds(r * BM, BM)

    @pl.when(phase == 0)
    def _stream():
        @pl.when(r == 0)
        def _zero():
            wb[:, 0:64] = jnp.zeros((N, 64), jnp.bfloat16)

        a_blk = a_ref[...]
        a16_blk = a_blk.astype(jnp.bfloat16)
        a16_s[rows, :] = a16_blk
        s = jnp.sum(a_blk, axis=1, keepdims=True) + 1.0
        d_blk = jax.lax.rsqrt(s)
        ws[rows, 96:97] = d_blk

        z = jnp.dot(x_ref[...], w1_ref[...], preferred_element_type=jnp.float32)
        n = jnp.sqrt(jnp.sum(z * z, axis=1, keepdims=True))
        zd1 = d_blk * (z / jnp.maximum(n, EPS))
        zd1b = zd1.astype(jnp.bfloat16)

        @pl.when(r < NBLK // 2)
        def _catch_half():
            acc_s[rows, :] = jnp.dot(
                a16_s[rows, pl.ds(0, N // 2)], wb[pl.ds(0, N // 2), 0:64],
                preferred_element_type=jnp.float32)

        @pl.when(r >= NBLK // 2)
        def _catch_full():
            acc_s[rows, :] = jnp.dot(
                a16_s[rows, :], wb[:, 0:64],
                preferred_element_type=jnp.float32)

        ws[rows, 0:64] = zd1
        wb[rows, 0:64] = zd1b

        acc_s[...] += jnp.dot(
            a16_s[:, pl.ds(r * BM, BM)], zd1b,
            preferred_element_type=jnp.float32)

        @pl.when(r == NBLK - 1)
        def _epilogue():
            d_all = ws[:, 96:97]
            h = jnp.maximum(SCALE * d_all * (acc_s[...] + ws[:, 0:64]), 0.0)
            g = jnp.dot(h, w2_ref[...], preferred_element_type=jnp.float32)
            gn = jnp.sqrt(jnp.sum(g * g, axis=1, keepdims=True))
            zd2 = d_all * (g / jnp.maximum(gn, EPS))
            ws[:, 64:80] = zd2
            wb[:, 64:80] = zd2.astype(jnp.bfloat16)

    @pl.when(phase == 1)
    def _conv2():
        d_blk = ws[rows, 96:97]
        acc = jnp.dot(a16_s[rows, :], wb[:, 64:80],
                      preferred_element_type=jnp.float32)
        ws[rows, 80:96] = SCALE * d_blk * (acc + ws[rows, 64:80])

    @pl.when(phase == 2)
    def _outer():
        p = jax.lax.dot_general(
            ws[rows, 80:96], ws[:, 80:96],
            (((1,), (1,)), ((), ())),
            preferred_element_type=jnp.float32,
        )
        o_ref[...] = 0.5 * jnp.tanh(0.5 * p) + 0.5


def kernel(A, X, W1, W2):
    return pl.pallas_call(
        _body,
        grid=(3 * NBLK,),
        in_specs=[
            pl.BlockSpec((BM, N), lambda i: (jnp.where(i < NBLK, i, NBLK - 1), 0)),
            pl.BlockSpec((BM, IN_FEAT), lambda i: (jnp.where(i < NBLK, i, NBLK - 1), 0)),
            pl.BlockSpec((IN_FEAT, HID), lambda i: (0, 0)),
            pl.BlockSpec((HID, LAT), lambda i: (0, 0)),
        ],
        out_specs=pl.BlockSpec(
            (BM, N), lambda i: (jnp.where(i >= 2 * NBLK, i % NBLK, 0), 0)
        ),
        out_shape=jax.ShapeDtypeStruct((N, N), jnp.float32),
        scratch_shapes=[
            pltpu.VMEM((N, N), jnp.bfloat16),
            pltpu.VMEM((N, 128), jnp.float32),
            pltpu.VMEM((N, 128), jnp.bfloat16),
            pltpu.VMEM((N, HID), jnp.float32),
        ],
        compiler_params=pltpu.CompilerParams(
            dimension_semantics=("arbitrary",),
        ),
    )(A, X, W1, W2)


# speedup vs baseline: 1.3772x; 1.0154x over previous
import jax
import jax.numpy as jnp
from jax.experimental import pallas as pl
from jax.experimental.pallas import tpu as pltpu

N = 4096
IN_FEAT = 128
HID = 64
LAT = 16
SCALE = 1.8
BM = 256
NBLK = N // BM
EPS = 1e-12


def _body(a_ref, x_ref, w1_ref, w2_ref, o_ref, a16_s, ws, wb, acc_s):
    i = pl.program_id(0)
    phase = i // NBLK
    r = i % NBLK
    rows = pl.ds(r * BM, BM)

    @pl.when(phase == 0)
    def _stream():
        a_blk = a_ref[...]
        a16_blk = a_blk.astype(jnp.bfloat16)
        a16_s[rows, :] = a16_blk
        s = jnp.sum(a_blk, axis=1, keepdims=True) + 1.0
        d_blk = jax.lax.rsqrt(s)
        ws[rows, 96:97] = d_blk

        z = jnp.dot(x_ref[...], w1_ref[...], preferred_element_type=jnp.float32)
        n = jnp.sqrt(jnp.sum(z * z, axis=1, keepdims=True))
        zd1 = d_blk * (z / jnp.maximum(n, EPS))
        zd1b = zd1.astype(jnp.bfloat16)

        @pl.when(r == 0)
        def _catch_none():
            acc_s[rows, :] = jnp.zeros((BM, HID), jnp.float32)

        for t in range(1, NBLK):
            @pl.when(r == t)
            def _catch(t=t):
                acc_s[rows, :] = jnp.dot(
                    a16_s[rows, pl.ds(0, t * BM)], wb[pl.ds(0, t * BM), 0:64],
                    preferred_element_type=jnp.float32)

        ws[rows, 0:64] = zd1
        wb[rows, 0:64] = zd1b

        acc_s[...] += jnp.dot(
            a16_s[:, pl.ds(r * BM, BM)], zd1b,
            preferred_element_type=jnp.float32)

        @pl.when(r == NBLK - 1)
        def _epilogue():
            d_all = ws[:, 96:97]
            h = jnp.maximum(SCALE * d_all * (acc_s[...] + ws[:, 0:64]), 0.0)
            g = jnp.dot(h, w2_ref[...], preferred_element_type=jnp.float32)
            gn = jnp.sqrt(jnp.sum(g * g, axis=1, keepdims=True))
            zd2 = d_all * (g / jnp.maximum(gn, EPS))
            ws[:, 64:80] = zd2
            wb[:, 64:80] = zd2.astype(jnp.bfloat16)

    @pl.when(phase == 1)
    def _conv2():
        d_blk = ws[rows, 96:97]
        acc = jnp.dot(a16_s[rows, :], wb[:, 64:80],
                      preferred_element_type=jnp.float32)
        ws[rows, 80:96] = SCALE * d_blk * (acc + ws[rows, 64:80])

    @pl.when(phase == 2)
    def _outer():
        p = jax.lax.dot_general(
            ws[rows, 80:96], ws[:, 80:96],
            (((1,), (1,)), ((), ())),
            preferred_element_type=jnp.float32,
        )
        o_ref[...] = 0.5 * jnp.tanh(0.5 * p) + 0.5


def kernel(A, X, W1, W2):
    return pl.pallas_call(
        _body,
        grid=(3 * NBLK,),
        in_specs=[
            pl.BlockSpec((BM, N), lambda i: (jnp.where(i < NBLK, i, NBLK - 1), 0)),
            pl.BlockSpec((BM, IN_FEAT), lambda i: (jnp.where(i < NBLK, i, NBLK - 1), 0)),
            pl.BlockSpec((IN_FEAT, HID), lambda i: (0, 0)),
            pl.BlockSpec((HID, LAT), lambda i: (0, 0)),
        ],
        out_specs=pl.BlockSpec(
            (BM, N), lambda i: (jnp.where(i >= 2 * NBLK, i % NBLK, 0), 0)
        ),
        out_shape=jax.ShapeDtypeStruct((N, N), jnp.float32),
        scratch_shapes=[
            pltpu.VMEM((N, N), jnp.bfloat16),
            pltpu.VMEM((N, 128), jnp.float32),
            pltpu.VMEM((N, 128), jnp.bfloat16),
            pltpu.VMEM((N, HID), jnp.float32),
        ],
        compiler_params=pltpu.CompilerParams(
            dimension_semantics=("arbitrary",),
        ),
    )(A, X, W1, W2)
